# Initial kernel scaffold; baseline (speedup 1.0000x reference)
import jax
import jax.numpy as jnp
from jax.experimental import pallas as pl


def kernel(x, adj1, adj2, deg, W1, b1, p1, W2, b2, p2, gW, att_s, att_d, gb, m1W, m1b, m2W, m2b, m3W, m3b):
    raise NotImplementedError("write your pallas kernel here")


# speedup vs baseline: 17.7812x; 17.7812x over previous
import functools

import jax
import jax.numpy as jnp
from jax import lax
from jax.experimental import pallas as pl
from jax.experimental.pallas import tpu as pltpu
from jax.experimental.pallas import tpu_sc as plsc

_N = 10000
_NP = 10240
_E = 640000
_EPAD = 655360
_NSUB = 16
_CHUNK = 512
_SUBC = 4
_NCH = _EPAD // (_NSUB * _CHUNK)
_EPT = _EPAD // _NSUB
_RPS = _NP // _NSUB
_BLK = 400
_NBLK = _N // _BLK


def _h0_body(x_ref, w_ref, o_ref):
    o_ref[0] = jnp.dot(x_ref[...], w_ref[...],
                       preferred_element_type=jnp.float32)


def _h0(x, W1):
    return pl.pallas_call(
        _h0_body,
        grid=(_NBLK, 2),
        in_specs=[
            pl.BlockSpec((_BLK, 128), lambda i, j: (i, 0)),
            pl.BlockSpec((128, 128), lambda i, j: (0, j)),
        ],
        out_specs=pl.BlockSpec((1, _BLK, 128), lambda i, j: (j, i, 0)),
        out_shape=jax.ShapeDtypeStruct((2, _N, 128), jnp.float32),
    )(x, W1)


def _gcn2_body(sa_ref, sb_ref, b1a_ref, b1b_ref, p_ref, w2a_ref, w2b_ref,
               o_ref):
    p = p_ref[0, 0]
    za = sa_ref[...] + b1a_ref[...]
    ta = jnp.maximum(jnp.where(za >= 0, za, p * za), 0.0)
    zb = sb_ref[...] + b1b_ref[...]
    tb = jnp.maximum(jnp.where(zb >= 0, zb, p * zb), 0.0)
    o_ref[...] = (
        jnp.dot(ta, w2a_ref[...], preferred_element_type=jnp.float32)
        + jnp.dot(tb, w2b_ref[...], preferred_element_type=jnp.float32))


def _gcn2(sa, sb, b1, p1, W2):
    b1a = b1[:128].reshape(1, 128)
    b1b = b1[128:].reshape(1, 128)
    return pl.pallas_call(
        _gcn2_body,
        grid=(_NBLK,),
        in_specs=[
            pl.BlockSpec((_BLK, 128), lambda i: (i, 0)),
            pl.BlockSpec((_BLK, 128), lambda i: (i, 0)),
            pl.BlockSpec((1, 128), lambda i: (0, 0)),
            pl.BlockSpec((1, 128), lambda i: (0, 0)),
            pl.BlockSpec(memory_space=pltpu.SMEM),
            pl.BlockSpec((128, 128), lambda i: (0, 0)),
            pl.BlockSpec((128, 128), lambda i: (0, 0)),
        ],
        out_specs=pl.BlockSpec((_BLK, 128), lambda i: (i, 0)),
        out_shape=jax.ShapeDtypeStruct((_N, 128), jnp.float32),
    )(sa, sb, b1a, b1b, p1.reshape(1, 1), W2[:128], W2[128:])


def _gat_prep_body(s2_ref, b2_ref, p_ref, gw_ref, atts_ref, attd_ref,
                   hs_ref, hh_ref, as_ref, ad_ref):
    p = p_ref[0, 0]
    z = s2_ref[...] + b2_ref[...]
    hs = jnp.where(z >= 0, z, p * z)
    hs_ref[...] = hs
    hh = jnp.dot(hs, gw_ref[...], preferred_element_type=jnp.float32)
    hh_ref[...] = hh
    as_ref[...] = jnp.sum(hh * atts_ref[...], axis=-1, keepdims=True)
    ad_ref[...] = jnp.sum(hh * attd_ref[...], axis=-1, keepdims=True)


def _gat_prep(s2, b2, p2, gW, att_s, att_d):
    return pl.pallas_call(
        _gat_prep_body,
        grid=(_NBLK,),
        in_specs=[
            pl.BlockSpec((_BLK, 128), lambda i: (i, 0)),
            pl.BlockSpec((1, 128), lambda i: (0, 0)),
            pl.BlockSpec(memory_space=pltpu.SMEM),
            pl.BlockSpec((128, 128), lambda i: (0, 0)),
            pl.BlockSpec((1, 128), lambda i: (0, 0)),
            pl.BlockSpec((1, 128), lambda i: (0, 0)),
        ],
        out_specs=[
            pl.BlockSpec((_BLK, 128), lambda i: (i, 0)),
            pl.BlockSpec((_BLK, 128), lambda i: (i, 0)),
            pl.BlockSpec((_BLK, 1), lambda i: (i, 0)),
            pl.BlockSpec((_BLK, 1), lambda i: (i, 0)),
        ],
        out_shape=[
            jax.ShapeDtypeStruct((_N, 128), jnp.float32),
            jax.ShapeDtypeStruct((_N, 128), jnp.float32),
            jax.ShapeDtypeStruct((_N, 1), jnp.float32),
            jax.ShapeDtypeStruct((_N, 1), jnp.float32),
        ],
    )(s2, b2.reshape(1, 128), p2.reshape(1, 1), gW,
      att_s.reshape(1, 128), att_d.reshape(1, 128))


def _final_body(hs_ref, hc_ref, u1_ref, u2_ref, s1_ref, s2_ref, deg_ref,
                gb_ref, m1w_ref, m1b_ref, m2w_ref, m2b_ref, w31_ref, w32_ref,
                sc_ref, hg_ref, beta_ref):
    hs = hs_ref[...]
    hc = hc_ref[...]
    hsg = jnp.maximum(u1_ref[...] / (s1_ref[...] + 1e-16) + gb_ref[...], 0.0)
    hcg = jnp.maximum(u2_ref[...] / (s2_ref[...] + 1e-16) + gb_ref[...], 0.0)
    w3d = sc_ref[0, 0]
    m3b = sc_ref[0, 1]
    degb = deg_ref[...].astype(jnp.bfloat16).astype(jnp.float32)
    w3db = w3d.astype(jnp.bfloat16).astype(jnp.float32)

    def mlp(a, b):
        z1 = jnp.dot(a, m1w_ref[...],
                     preferred_element_type=jnp.float32) + m1b_ref[...]
        z2 = jnp.dot(b, m2w_ref[...],
                     preferred_element_type=jnp.float32) + m2b_ref[...]
        logit = (jnp.dot(z1, w31_ref[...],
                         preferred_element_type=jnp.float32)
                 + jnp.dot(z2, w32_ref[...],
                           preferred_element_type=jnp.float32)
                 + degb * w3db + m3b)
        return jax.nn.sigmoid(logit)

    beta_ref[...] = mlp(hs, hc)
    beta_g = mlp(hsg, hcg)
    hg_ref[...] = hsg + beta_g * hcg


def _final(hs, hc, u1, u2, s1, s2, deg, gb, m1W, m1b, m2W, m2b, m3W, m3b):
    sc = jnp.stack([m3W[128, 0], m3b[0]]).reshape(1, 2)
    return pl.pallas_call(
        _final_body,
        grid=(_NBLK,),
        in_specs=[
            pl.BlockSpec((_BLK, 128), lambda i: (i, 0)),
            pl.BlockSpec((_BLK, 128), lambda i: (i, 0)),
            pl.BlockSpec((_BLK, 128), lambda i: (i, 0)),
            pl.BlockSpec((_BLK, 128), lambda i: (i, 0)),
            pl.BlockSpec((_BLK, 1), lambda i: (i, 0)),
            pl.BlockSpec((_BLK, 1), lambda i: (i, 0)),
            pl.BlockSpec((_BLK, 1), lambda i: (i, 0)),
            pl.BlockSpec((1, 128), lambda i: (0, 0)),
            pl.BlockSpec((128, 64), lambda i: (0, 0)),
            pl.BlockSpec((1, 64), lambda i: (0, 0)),
            pl.BlockSpec((128, 64), lambda i: (0, 0)),
            pl.BlockSpec((1, 64), lambda i: (0, 0)),
            pl.BlockSpec((64, 1), lambda i: (0, 0)),
            pl.BlockSpec((64, 1), lambda i: (0, 0)),
            pl.BlockSpec(memory_space=pltpu.SMEM),
        ],
        out_specs=[
            pl.BlockSpec((_BLK, 128), lambda i: (i, 0)),
            pl.BlockSpec((_BLK, 1), lambda i: (i, 0)),
        ],
        out_shape=[
            jax.ShapeDtypeStruct((_N, 128), jnp.float32),
            jax.ShapeDtypeStruct((_N, 1), jnp.float32),
        ],
    )(hs, hc, u1, u2, s1, s2, deg, gb.reshape(1, 128), m1W,
      m1b.reshape(1, 64), m2W, m2b.reshape(1, 64),
      m3W[:64], m3W[64:128], sc)


def _zero_rows(rows):
    zv = jnp.zeros((16,), jnp.float32)

    def body(i, carry):
        rows[i // 8, pl.ds((i % 8) * 16, 16)] = zv
        return carry

    lax.fori_loop(0, 1024, body, 0)


def _segsum_body(h, srcr, dstr, out, sidx, didx, rows, acc, sem):
    c = lax.axis_index("c")
    s = lax.axis_index("s")
    _zero_rows(rows)
    for r in range(_RPS // 128):
        pltpu.sync_copy(rows, acc.at[pl.ds(s * _RPS + r * 128, 128)])
    plsc.subcore_barrier()

    def chunk_body(i, carry):
        row0 = (s * _NCH + i) * _SUBC
        pltpu.sync_copy(srcr.at[c, pl.ds(row0, _SUBC)], sidx)
        pltpu.sync_copy(dstr.at[c, pl.ds(row0, _SUBC)], didx)
        for j in range(_SUBC):
            pltpu.async_copy(h.at[sidx.at[j]], rows, sem).wait()
            pltpu.sync_copy(rows, acc.at[didx.at[j]], add=True)
        return carry

    lax.fori_loop(0, _NCH, chunk_body, 0)
    plsc.subcore_barrier()
    for r in range(_RPS // 128):
        pltpu.sync_copy(acc.at[pl.ds(s * _RPS + r * 128, 128)], rows)
        pltpu.sync_copy(rows, out.at[pl.ds(c * _NP + s * _RPS + r * 128, 128)])


@functools.cache
def _sc_kernels():
    mesh = plsc.VectorSubcoreMesh(core_axis_name="c", subcore_axis_name="s")
    params = pltpu.CompilerParams(needs_layout_passes=False)
    segsum = pl.kernel(
        _segsum_body,
        out_type=jax.ShapeDtypeStruct((2 * _NP, 128), jnp.float32),
        mesh=mesh,
        compiler_params=params,
        scratch_types=[
            pltpu.VMEM((_SUBC, 128), jnp.int32),
            pltpu.VMEM((_SUBC, 128), jnp.int32),
            pltpu.VMEM((128, 128), jnp.float32),
            pltpu.VMEM_SHARED((_NP, 128), jnp.float32),
            pltpu.SemaphoreType.DMA,
        ],
    )
    scores = pl.kernel(
        _gat_scores_body,
        out_type=(
            jax.ShapeDtypeStruct((2, _NP), jnp.float32),
            jax.ShapeDtypeStruct((2, _NSUB * _NCH * _SUBC, 128),
                                 jnp.float32),
            jax.ShapeDtypeStruct((2, 16, _NP), jnp.float32),
        ),
        mesh=mesh,
        compiler_params=params,
        scratch_types=[
            pltpu.VMEM((_SUBC, 128), jnp.int32),
            pltpu.VMEM((_SUBC, 128), jnp.int32),
            pltpu.VMEM((_NP,), jnp.float32),
            pltpu.VMEM((_NP,), jnp.float32),
            pltpu.VMEM((_NP,), jnp.float32),
            pltpu.VMEM((128,), jnp.float32),
            pltpu.VMEM((_RPS,), jnp.float32),
            pltpu.VMEM((_RPS,), jnp.float32),
            pltpu.VMEM_SHARED((_NP,), jnp.float32),
        ],
    )
    wsum = pl.kernel(
        _wsum_body,
        out_type=jax.ShapeDtypeStruct((2 * _NP, 128), jnp.float32),
        mesh=mesh,
        compiler_params=params,
        scratch_types=[
            pltpu.VMEM((_SUBC, 128), jnp.int32),
            pltpu.VMEM((_SUBC, 128), jnp.int32),
            pltpu.VMEM((128, 128), jnp.float32),
            pltpu.VMEM((128,), jnp.float32),
            pltpu.VMEM_SHARED((_NP, 128), jnp.float32),
            pltpu.SemaphoreType.DMA,
        ],
    )
    return segsum, scores, wsum


def _gat_scores_body(asc, adc, srcr, dstr, sout, exv, mst,
                     sidx, didx, asb, adb, mb, exb, zs, macc, sden):
    c = lax.axis_index("c")
    s = lax.axis_index("s")
    coff = c * _N
    zv = jnp.zeros((16,), jnp.float32)

    def zbody(i, carry):
        zs[pl.ds(i * 16, 16)] = zv
        return carry

    lax.fori_loop(0, _RPS // 16, zbody, 0)
    pltpu.sync_copy(zs, sden.at[pl.ds(s * _RPS, _RPS)])
    ninf = jnp.full((16,), -3e38, jnp.float32)

    def mbody(i, carry):
        mb[pl.ds(i * 16, 16)] = ninf
        return carry

    lax.fori_loop(0, _NP // 16, mbody, 0)
    pltpu.sync_copy(asc.at[c], asb)
    pltpu.sync_copy(adc.at[c], adb)

    def edge_e(i, j, k):
        sv = sidx[j, pl.ds(k * 16, 16)] - coff
        dv = didx[j, pl.ds(k * 16, 16)]
        e = plsc.load_gather(asb, [sv]) + plsc.load_gather(adb, [dv])
        return jnp.where(e >= 0, e, 0.2 * e), dv

    def p1_chunk(i, carry):
        row0 = (s * _NCH + i) * _SUBC
        pltpu.sync_copy(srcr.at[c, pl.ds(row0, _SUBC)], sidx)
        pltpu.sync_copy(dstr.at[c, pl.ds(row0, _SUBC)], didx)
        for j in range(_SUBC):
            def grp(k, carry2):
                e, dv = edge_e(i, j, k)

                def cond(pend):
                    return jnp.any(pend)

                def body(pend):
                    cur = plsc.load_gather(mb, [dv])
                    nm = jnp.maximum(cur, e)
                    plsc.store_scatter(mb, [dv], nm, mask=pend)
                    chk = plsc.load_gather(mb, [dv])
                    return chk < e

                lax.while_loop(cond, body, jnp.ones((16,), jnp.bool_))
                return carry2

            lax.fori_loop(0, 8, grp, 0)
        return carry

    lax.fori_loop(0, _NCH, p1_chunk, 0)

    pltpu.sync_copy(mb, mst.at[c, s])
    plsc.subcore_barrier()
    sl = pl.ds(s * _RPS, _RPS)
    pltpu.sync_copy(mst.at[c, 0, sl], macc)

    def redmax(r, carry):
        pltpu.sync_copy(mst.at[c, r, sl], zs)

        def vmax(i, carry2):
            v = pl.ds(i * 16, 16)
            macc[v] = jnp.maximum(macc[v], zs[v])
            return carry2

        lax.fori_loop(0, _RPS // 16, vmax, 0)
        return carry

    lax.fori_loop(1, 16, redmax, 0)

    def fin(i, carry):
        v = pl.ds(i * 16, 16)
        m = macc[v]
        macc[v] = jnp.where(m < -2.9e38, 0.0, m)
        return carry

    lax.fori_loop(0, _RPS // 16, fin, 0)
    pltpu.sync_copy(macc, mst.at[c, 0, sl])
    plsc.subcore_barrier()
    pltpu.sync_copy(mst.at[c, 0], mb)

    def p3_chunk(i, carry):
        row0 = (s * _NCH + i) * _SUBC
        pltpu.sync_copy(srcr.at[c, pl.ds(row0, _SUBC)], sidx)
        pltpu.sync_copy(dstr.at[c, pl.ds(row0, _SUBC)], didx)
        for j in range(_SUBC):
            def grp(k, carry2):
                e, dv = edge_e(i, j, k)
                mg = plsc.load_gather(mb, [dv])
                exb[pl.ds(k * 16, 16)] = jnp.exp(e - mg)
                return carry2

            lax.fori_loop(0, 8, grp, 0)
            pltpu.sync_copy(exb, exv.at[c, row0 + j])
            pltpu.sync_copy(exb, sden.at[didx.at[j]], add=True)
        return carry

    lax.fori_loop(0, _NCH, p3_chunk, 0)
    plsc.subcore_barrier()
    pltpu.sync_copy(sden.at[pl.ds(s * _RPS, _RPS)], zs)
    pltpu.sync_copy(zs, sout.at[c, pl.ds(s * _RPS, _RPS)])


def _wsum_body(h, srcr, dstr, exv, out, sidx, didx, rows, exb, acc, sem):
    c = lax.axis_index("c")
    s = lax.axis_index("s")
    _zero_rows(rows)
    for r in range(_RPS // 128):
        pltpu.sync_copy(rows, acc.at[pl.ds(s * _RPS + r * 128, 128)])
    plsc.subcore_barrier()

    def chunk_body(i, carry):
        row0 = (s * _NCH + i) * _SUBC
        pltpu.sync_copy(srcr.at[c, pl.ds(row0, _SUBC)], sidx)
        pltpu.sync_copy(dstr.at[c, pl.ds(row0, _SUBC)], didx)
        for j in range(_SUBC):
            pltpu.sync_copy(exv.at[c, row0 + j], exb)
            pltpu.async_copy(h.at[sidx.at[j]], rows, sem).wait()

            def scale(k, carry2):
                w = plsc.load_gather(exb, [jnp.zeros((16,), jnp.int32) + k])
                for f in range(8):
                    sl = pl.ds(f * 16, 16)
                    rows[k, sl] = rows[k, sl] * w
                return carry2

            lax.fori_loop(0, 128, scale, 0)
            pltpu.sync_copy(rows, acc.at[didx.at[j]], add=True)
        return carry

    lax.fori_loop(0, _NCH, chunk_body, 0)
    plsc.subcore_barrier()
    for r in range(_RPS // 128):
        pltpu.sync_copy(acc.at[pl.ds(s * _RPS + r * 128, 128)], rows)
        pltpu.sync_copy(rows, out.at[pl.ds(c * _NP + s * _RPS + r * 128, 128)])


def kernel(x, adj1, adj2, deg, W1, b1, p1, W2, b2, p2, gW, att_s, att_d, gb,
           m1W, m1b, m2W, m2b, m3W, m3b):
    npad = _EPAD - _E
    pad_src = jnp.arange(npad, dtype=jnp.int32) % _N
    pad_dst = _N + jnp.arange(npad, dtype=jnp.int32) % (_NP - _N)

    def epack(v, pad):
        return jnp.concatenate(
            [v.astype(jnp.int32), pad]).reshape(_NSUB * _NCH * _SUBC, 128)

    s1p = epack(adj1[0], pad_src)
    d1p = epack(adj1[1], pad_dst)
    s2p = epack(adj2[0], pad_src)
    d2p = epack(adj2[1], pad_dst)

    _segsum_kernel, _gat_scores, _wsum = _sc_kernels()
    H0 = _h0(x, W1).reshape(2 * _N, 128)

    S1 = _segsum_kernel(H0, jnp.stack([s1p, s1p + _N]),
                        jnp.stack([d1p, d1p]))
    T1 = _segsum_kernel(H0, jnp.stack([s2p, s2p + _N]),
                        jnp.stack([d2p, d2p]))
    H1_1 = _gcn2(S1[:_N], S1[_NP:_NP + _N], b1, p1, W2)
    H1_2 = _gcn2(T1[:_N], T1[_NP:_NP + _N], b1, p1, W2)

    H1 = jnp.concatenate([H1_1, H1_2], axis=0)
    srcb = jnp.stack([s1p, s2p + _N])
    dstb = jnp.stack([d1p, d2p])
    S2 = _segsum_kernel(H1, srcb, dstb)

    hs1, HH1, AS1, AD1 = _gat_prep(S2[:_N], b2, p2, gW, att_s, att_d)
    hs2, HH2, AS2, AD2 = _gat_prep(S2[_NP:_NP + _N], b2, p2, gW, att_s,
                                   att_d)

    def pad_np(a):
        return jnp.pad(a.reshape(_N), (0, _NP - _N))

    SD, EXV, _ = _gat_scores(jnp.stack([pad_np(AS1), pad_np(AS2)]),
                             jnp.stack([pad_np(AD1), pad_np(AD2)]),
                             srcb, dstb)
    U = _wsum(jnp.concatenate([HH1, HH2], axis=0), srcb, dstb, EXV)

    hg, beta = _final(hs1, hs2, U[:_N], U[_NP:_NP + _N],
                      SD[0, :_N].reshape(_N, 1), SD[1, :_N].reshape(_N, 1),
                      deg, gb, m1W, m1b, m2W, m2b, m3W, m3b)
    return (hs1, hs2, hg, beta)


# speedup vs baseline: 28.3496x; 1.5944x over previous
import functools

import jax
import jax.numpy as jnp
from jax import lax
from jax.experimental import pallas as pl
from jax.experimental.pallas import tpu as pltpu
from jax.experimental.pallas import tpu_sc as plsc

_N = 10000
_NP = 10240
_E = 640000
_EPAD = 655360
_NSUB = 16
_CHUNK = 512
_SUBC = 4
_NCH = _EPAD // (_NSUB * _CHUNK)
_EPT = _EPAD // _NSUB
_RPS = _NP // _NSUB
_BLK = 400
_NBLK = _N // _BLK


def _h0_body(x_ref, w_ref, o_ref):
    o_ref[0] = jnp.dot(x_ref[...], w_ref[...],
                       preferred_element_type=jnp.float32)


def _h0(x, W1):
    return pl.pallas_call(
        _h0_body,
        grid=(_NBLK, 2),
        in_specs=[
            pl.BlockSpec((_BLK, 128), lambda i, j: (i, 0)),
            pl.BlockSpec((128, 128), lambda i, j: (0, j)),
        ],
        out_specs=pl.BlockSpec((1, _BLK, 128), lambda i, j: (j, i, 0)),
        out_shape=jax.ShapeDtypeStruct((2, _N, 128), jnp.float32),
    )(x, W1)


def _gcn2_body(sa_ref, sb_ref, b1a_ref, b1b_ref, p_ref, w2a_ref, w2b_ref,
               o_ref):
    p = p_ref[0, 0]
    za = sa_ref[...] + b1a_ref[...]
    ta = jnp.maximum(jnp.where(za >= 0, za, p * za), 0.0)
    zb = sb_ref[...] + b1b_ref[...]
    tb = jnp.maximum(jnp.where(zb >= 0, zb, p * zb), 0.0)
    o_ref[...] = (
        jnp.dot(ta, w2a_ref[...], preferred_element_type=jnp.float32)
        + jnp.dot(tb, w2b_ref[...], preferred_element_type=jnp.float32))


def _gcn2(sa, sb, b1, p1, W2):
    b1a = b1[:128].reshape(1, 128)
    b1b = b1[128:].reshape(1, 128)
    return pl.pallas_call(
        _gcn2_body,
        grid=(_NBLK,),
        in_specs=[
            pl.BlockSpec((_BLK, 128), lambda i: (i, 0)),
            pl.BlockSpec((_BLK, 128), lambda i: (i, 0)),
            pl.BlockSpec((1, 128), lambda i: (0, 0)),
            pl.BlockSpec((1, 128), lambda i: (0, 0)),
            pl.BlockSpec(memory_space=pltpu.SMEM),
            pl.BlockSpec((128, 128), lambda i: (0, 0)),
            pl.BlockSpec((128, 128), lambda i: (0, 0)),
        ],
        out_specs=pl.BlockSpec((_BLK, 128), lambda i: (i, 0)),
        out_shape=jax.ShapeDtypeStruct((_N, 128), jnp.float32),
    )(sa, sb, b1a, b1b, p1.reshape(1, 1), W2[:128], W2[128:])


def _gat_prep_body(s2_ref, b2_ref, p_ref, gw_ref, atts_ref, attd_ref,
                   hs_ref, hh_ref, as_ref, ad_ref):
    p = p_ref[0, 0]
    z = s2_ref[...] + b2_ref[...]
    hs = jnp.where(z >= 0, z, p * z)
    hs_ref[...] = hs
    hh = jnp.dot(hs, gw_ref[...], preferred_element_type=jnp.float32)
    hh_ref[...] = hh
    as_ref[...] = jnp.sum(hh * atts_ref[...], axis=-1, keepdims=True)
    ad_ref[...] = jnp.sum(hh * attd_ref[...], axis=-1, keepdims=True)


def _gat_prep(s2, b2, p2, gW, att_s, att_d):
    return pl.pallas_call(
        _gat_prep_body,
        grid=(_NBLK,),
        in_specs=[
            pl.BlockSpec((_BLK, 128), lambda i: (i, 0)),
            pl.BlockSpec((1, 128), lambda i: (0, 0)),
            pl.BlockSpec(memory_space=pltpu.SMEM),
            pl.BlockSpec((128, 128), lambda i: (0, 0)),
            pl.BlockSpec((1, 128), lambda i: (0, 0)),
            pl.BlockSpec((1, 128), lambda i: (0, 0)),
        ],
        out_specs=[
            pl.BlockSpec((_BLK, 128), lambda i: (i, 0)),
            pl.BlockSpec((_BLK, 128), lambda i: (i, 0)),
            pl.BlockSpec((_BLK, 1), lambda i: (i, 0)),
            pl.BlockSpec((_BLK, 1), lambda i: (i, 0)),
        ],
        out_shape=[
            jax.ShapeDtypeStruct((_N, 128), jnp.float32),
            jax.ShapeDtypeStruct((_N, 128), jnp.float32),
            jax.ShapeDtypeStruct((_N, 1), jnp.float32),
            jax.ShapeDtypeStruct((_N, 1), jnp.float32),
        ],
    )(s2, b2.reshape(1, 128), p2.reshape(1, 1), gW,
      att_s.reshape(1, 128), att_d.reshape(1, 128))


def _final_body(hs_ref, hc_ref, u1_ref, u2_ref, s1_ref, s2_ref, deg_ref,
                gb_ref, m1w_ref, m1b_ref, m2w_ref, m2b_ref, w31_ref, w32_ref,
                sc_ref, hg_ref, beta_ref):
    hs = hs_ref[...]
    hc = hc_ref[...]
    hsg = jnp.maximum(u1_ref[...] / (s1_ref[...] + 1e-16) + gb_ref[...], 0.0)
    hcg = jnp.maximum(u2_ref[...] / (s2_ref[...] + 1e-16) + gb_ref[...], 0.0)
    w3d = sc_ref[0, 0]
    m3b = sc_ref[0, 1]
    degb = deg_ref[...].astype(jnp.bfloat16).astype(jnp.float32)
    w3db = w3d.astype(jnp.bfloat16).astype(jnp.float32)

    def mlp(a, b):
        z1 = jnp.dot(a, m1w_ref[...],
                     preferred_element_type=jnp.float32) + m1b_ref[...]
        z2 = jnp.dot(b, m2w_ref[...],
                     preferred_element_type=jnp.float32) + m2b_ref[...]
        logit = (jnp.dot(z1, w31_ref[...],
                         preferred_element_type=jnp.float32)
                 + jnp.dot(z2, w32_ref[...],
                           preferred_element_type=jnp.float32)
                 + degb * w3db + m3b)
        return jax.nn.sigmoid(logit)

    beta_ref[...] = mlp(hs, hc)
    beta_g = mlp(hsg, hcg)
    hg_ref[...] = hsg + beta_g * hcg


def _final(hs, hc, u1, u2, s1, s2, deg, gb, m1W, m1b, m2W, m2b, m3W, m3b):
    sc = jnp.stack([m3W[128, 0], m3b[0]]).reshape(1, 2)
    return pl.pallas_call(
        _final_body,
        grid=(_NBLK,),
        in_specs=[
            pl.BlockSpec((_BLK, 128), lambda i: (i, 0)),
            pl.BlockSpec((_BLK, 128), lambda i: (i, 0)),
            pl.BlockSpec((_BLK, 128), lambda i: (i, 0)),
            pl.BlockSpec((_BLK, 128), lambda i: (i, 0)),
            pl.BlockSpec((_BLK, 1), lambda i: (i, 0)),
            pl.BlockSpec((_BLK, 1), lambda i: (i, 0)),
            pl.BlockSpec((_BLK, 1), lambda i: (i, 0)),
            pl.BlockSpec((1, 128), lambda i: (0, 0)),
            pl.BlockSpec((128, 64), lambda i: (0, 0)),
            pl.BlockSpec((1, 64), lambda i: (0, 0)),
            pl.BlockSpec((128, 64), lambda i: (0, 0)),
            pl.BlockSpec((1, 64), lambda i: (0, 0)),
            pl.BlockSpec((64, 1), lambda i: (0, 0)),
            pl.BlockSpec((64, 1), lambda i: (0, 0)),
            pl.BlockSpec(memory_space=pltpu.SMEM),
        ],
        out_specs=[
            pl.BlockSpec((_BLK, 128), lambda i: (i, 0)),
            pl.BlockSpec((_BLK, 1), lambda i: (i, 0)),
        ],
        out_shape=[
            jax.ShapeDtypeStruct((_N, 128), jnp.float32),
            jax.ShapeDtypeStruct((_N, 1), jnp.float32),
        ],
    )(hs, hc, u1, u2, s1, s2, deg, gb.reshape(1, 128), m1W,
      m1b.reshape(1, 64), m2W, m2b.reshape(1, 64),
      m3W[:64], m3W[64:128], sc)


def _zero_rows(rows):
    zv = jnp.zeros((16,), jnp.float32)

    def body(i, carry):
        rows[i // 8, pl.ds((i % 8) * 16, 16)] = zv
        return carry

    lax.fori_loop(0, 1024, body, 0)


_NB = _NCH * _SUBC


def _gather_scatter_pipeline(c, s, h, srcr, dstr, acc, sidx3, didx3, rows2,
                             gsem, ssem, pxs, pxd, exbf=None, exv=None,
                             pxe=None):
    base = s * _NB
    zk = jnp.zeros((16,), jnp.int32)
    pltpu.sync_copy(srcr.at[c, base], sidx3.at[0])
    pltpu.sync_copy(dstr.at[c, base], didx3.at[0])
    pltpu.async_copy(h.at[sidx3.at[0]], rows2.at[0], gsem.at[0])
    pltpu.async_copy(srcr.at[c, base + 1], sidx3.at[1], pxs.at[1])
    pltpu.async_copy(dstr.at[c, base + 1], didx3.at[1], pxd.at[1])
    if exbf is not None:
        pltpu.sync_copy(exv.at[c, base], exbf.at[pl.ds(0, 128)])
        pltpu.async_copy(exv.at[c, base + 1], exbf.at[pl.ds(128, 128)],
                         pxe.at[1])

    def body(j, carry):
        par = lax.rem(j, 2)
        nxt = 1 - par
        i3 = lax.rem(j, 3)
        n3 = lax.rem(j + 1, 3)
        p3 = lax.rem(j + 2, 3)
        pltpu.make_async_copy(h.at[sidx3.at[i3]], rows2.at[par],
                              gsem.at[par]).wait()

        @pl.when(j + 1 < _NB)
        def _issue_next_gather():
            pltpu.make_async_copy(srcr.at[c, base], sidx3.at[n3],
                                  pxs.at[n3]).wait()
            pltpu.make_async_copy(dstr.at[c, base], didx3.at[n3],
                                  pxd.at[n3]).wait()
            if exbf is not None:
                pltpu.make_async_copy(exv.at[c, base],
                                      exbf.at[pl.ds(0, 128)],
                                      pxe.at[n3]).wait()

            @pl.when(j >= 1)
            def _drain_prev_scatter():
                pltpu.make_async_copy(rows2.at[nxt], acc.at[didx3.at[i3]],
                                      ssem.at[nxt]).wait()

            pltpu.async_copy(h.at[sidx3.at[n3]], rows2.at[nxt],
                             gsem.at[nxt])

        if exbf is not None:
            off = i3 * 128

            def scale(k, carry2):
                k2 = k * 2
                w0 = plsc.load_gather(exbf, [zk + (off + k2)])
                w1 = plsc.load_gather(exbf, [zk + (off + k2 + 1)])
                for f in range(8):
                    sl = pl.ds(f * 16, 16)
                    rows2[par, k2, sl] = rows2[par, k2, sl] * w0
                    rows2[par, k2 + 1, sl] = rows2[par, k2 + 1, sl] * w1
                return carry2

            lax.fori_loop(0, 64, scale, 0)

        pltpu.make_async_copy(rows2.at[par], acc.at[didx3.at[i3]],
                              ssem.at[par]).start(add=True)

        @pl.when(j + 2 < _NB)
        def _prefetch_idx():
            pltpu.async_copy(srcr.at[c, base + j + 2], sidx3.at[p3],
                             pxs.at[p3])
            pltpu.async_copy(dstr.at[c, base + j + 2], didx3.at[p3],
                             pxd.at[p3])
            if exbf is not None:
                pltpu.async_copy(exv.at[c, base + j + 2],
                                 exbf.at[pl.ds(p3 * 128, 128)], pxe.at[p3])

        return carry

    lax.fori_loop(0, _NB, body, 0)
    for par in range(2):
        pltpu.make_async_copy(rows2.at[par], acc.at[didx3.at[0]],
                              ssem.at[par]).wait()


def _acc_zero_and_loop(c, s, h, srcr, dstr, out, sidx3, didx3, rows2, acc,
                       gsem, ssem, pxs, pxd, exbf=None, exv=None, pxe=None):
    rows0 = rows2.at[0]
    _zero_rows(rows0)
    for r in range(_RPS // 128):
        pltpu.sync_copy(rows0, acc.at[pl.ds(s * _RPS + r * 128, 128)])
    plsc.subcore_barrier()
    _gather_scatter_pipeline(c, s, h, srcr, dstr, acc, sidx3, didx3, rows2,
                             gsem, ssem, pxs, pxd, exbf, exv, pxe)
    plsc.subcore_barrier()
    for r in range(_RPS // 128):
        pltpu.sync_copy(acc.at[pl.ds(s * _RPS + r * 128, 128)], rows0)
        pltpu.sync_copy(rows0,
                        out.at[pl.ds(c * _NP + s * _RPS + r * 128, 128)])


def _segsum_body(h, srcr, dstr, out, sidx3, didx3, rows2, acc, gsem, ssem,
                 pxs, pxd):
    c = lax.axis_index("c")
    s = lax.axis_index("s")
    _acc_zero_and_loop(c, s, h, srcr, dstr, out, sidx3, didx3, rows2, acc,
                       gsem, ssem, pxs, pxd)


@functools.cache
def _sc_kernels():
    mesh = plsc.VectorSubcoreMesh(core_axis_name="c", subcore_axis_name="s")
    params = pltpu.CompilerParams(needs_layout_passes=False)
    pipe_scratch = [
        pltpu.VMEM((3, 128), jnp.int32),
        pltpu.VMEM((3, 128), jnp.int32),
        pltpu.VMEM((2, 128, 128), jnp.float32),
        pltpu.VMEM_SHARED((_NP, 128), jnp.float32),
        pltpu.SemaphoreType.DMA((2,)),
        pltpu.SemaphoreType.DMA((2,)),
        pltpu.SemaphoreType.DMA((3,)),
        pltpu.SemaphoreType.DMA((3,)),
    ]
    segsum = pl.kernel(
        _segsum_body,
        out_type=jax.ShapeDtypeStruct((2 * _NP, 128), jnp.float32),
        mesh=mesh,
        compiler_params=params,
        scratch_types=pipe_scratch,
    )
    scores = pl.kernel(
        _gat_scores_body,
        out_type=(
            jax.ShapeDtypeStruct((2, _NP), jnp.float32),
            jax.ShapeDtypeStruct((2, _NSUB * _NCH * _SUBC, 128),
                                 jnp.float32),
            jax.ShapeDtypeStruct((2, 16, _NP), jnp.float32),
        ),
        mesh=mesh,
        compiler_params=params,
        scratch_types=[
            pltpu.VMEM((_SUBC, 128), jnp.int32),
            pltpu.VMEM((_SUBC, 128), jnp.int32),
            pltpu.VMEM((_NP,), jnp.float32),
            pltpu.VMEM((_NP,), jnp.float32),
            pltpu.VMEM((_NP,), jnp.float32),
            pltpu.VMEM((128,), jnp.float32),
            pltpu.VMEM((_RPS,), jnp.float32),
            pltpu.VMEM((_RPS,), jnp.float32),
            pltpu.VMEM_SHARED((_NP,), jnp.float32),
        ],
    )
    wsum = pl.kernel(
        _wsum_body,
        out_type=jax.ShapeDtypeStruct((2 * _NP, 128), jnp.float32),
        mesh=mesh,
        compiler_params=params,
        scratch_types=pipe_scratch[:3] + [
            pltpu.VMEM((384,), jnp.float32),
        ] + pipe_scratch[3:] + [
            pltpu.SemaphoreType.DMA((3,)),
        ],
    )
    return segsum, scores, wsum


def _gat_scores_body(asc, adc, srcr, dstr, sout, exv, mst,
                     sidx, didx, asb, adb, mb, exb, zs, macc, sden):
    c = lax.axis_index("c")
    s = lax.axis_index("s")
    coff = c * _N
    zv = jnp.zeros((16,), jnp.float32)

    def zbody(i, carry):
        zs[pl.ds(i * 16, 16)] = zv
        return carry

    lax.fori_loop(0, _RPS // 16, zbody, 0)
    pltpu.sync_copy(zs, sden.at[pl.ds(s * _RPS, _RPS)])
    ninf = jnp.full((16,), -3e38, jnp.float32)

    def mbody(i, carry):
        mb[pl.ds(i * 16, 16)] = ninf
        return carry

    lax.fori_loop(0, _NP // 16, mbody, 0)
    pltpu.sync_copy(asc.at[c], asb)
    pltpu.sync_copy(adc.at[c], adb)

    def edge_e(i, j, k):
        sv = sidx[j, pl.ds(k * 16, 16)] - coff
        dv = didx[j, pl.ds(k * 16, 16)]
        e = plsc.load_gather(asb, [sv]) + plsc.load_gather(adb, [dv])
        return jnp.where(e >= 0, e, 0.2 * e), dv

    def p1_chunk(i, carry):
        row0 = (s * _NCH + i) * _SUBC
        pltpu.sync_copy(srcr.at[c, pl.ds(row0, _SUBC)], sidx)
        pltpu.sync_copy(dstr.at[c, pl.ds(row0, _SUBC)], didx)
        for j in range(_SUBC):
            def grp(k, carry2):
                e, dv = edge_e(i, j, k)

                def cond(pend):
                    return jnp.any(pend)

                def body(pend):
                    cur = plsc.load_gather(mb, [dv])
                    nm = jnp.maximum(cur, e)
                    plsc.store_scatter(mb, [dv], nm, mask=pend)
                    chk = plsc.load_gather(mb, [dv])
                    return chk < e

                lax.while_loop(cond, body, jnp.ones((16,), jnp.bool_))
                return carry2

            lax.fori_loop(0, 8, grp, 0)
        return carry

    lax.fori_loop(0, _NCH, p1_chunk, 0)

    pltpu.sync_copy(mb, mst.at[c, s])
    plsc.subcore_barrier()
    sl = pl.ds(s * _RPS, _RPS)
    pltpu.sync_copy(mst.at[c, 0, sl], macc)

    def redmax(r, carry):
        pltpu.sync_copy(mst.at[c, r, sl], zs)

        def vmax(i, carry2):
            v = pl.ds(i * 16, 16)
            macc[v] = jnp.maximum(macc[v], zs[v])
            return carry2

        lax.fori_loop(0, _RPS // 16, vmax, 0)
        return carry

    lax.fori_loop(1, 16, redmax, 0)

    def fin(i, carry):
        v = pl.ds(i * 16, 16)
        m = macc[v]
        macc[v] = jnp.where(m < -2.9e38, 0.0, m)
        return carry

    lax.fori_loop(0, _RPS // 16, fin, 0)
    pltpu.sync_copy(macc, mst.at[c, 0, sl])
    plsc.subcore_barrier()
    pltpu.sync_copy(mst.at[c, 0], mb)

    def p3_chunk(i, carry):
        row0 = (s * _NCH + i) * _SUBC
        pltpu.sync_copy(srcr.at[c, pl.ds(row0, _SUBC)], sidx)
        pltpu.sync_copy(dstr.at[c, pl.ds(row0, _SUBC)], didx)
        for j in range(_SUBC):
            def grp(k, carry2):
                e, dv = edge_e(i, j, k)
                mg = plsc.load_gather(mb, [dv])
                exb[pl.ds(k * 16, 16)] = jnp.exp(e - mg)
                return carry2

            lax.fori_loop(0, 8, grp, 0)
            pltpu.sync_copy(exb, exv.at[c, row0 + j])
            pltpu.sync_copy(exb, sden.at[didx.at[j]], add=True)
        return carry

    lax.fori_loop(0, _NCH, p3_chunk, 0)
    plsc.subcore_barrier()
    pltpu.sync_copy(sden.at[pl.ds(s * _RPS, _RPS)], zs)
    pltpu.sync_copy(zs, sout.at[c, pl.ds(s * _RPS, _RPS)])


def _wsum_body(h, srcr, dstr, exv, out, sidx3, didx3, rows2, exbf, acc,
               gsem, ssem, pxs, pxd, pxe):
    c = lax.axis_index("c")
    s = lax.axis_index("s")
    _acc_zero_and_loop(c, s, h, srcr, dstr, out, sidx3, didx3, rows2, acc,
                       gsem, ssem, pxs, pxd, exbf, exv, pxe)


def kernel(x, adj1, adj2, deg, W1, b1, p1, W2, b2, p2, gW, att_s, att_d, gb,
           m1W, m1b, m2W, m2b, m3W, m3b):
    npad = _EPAD - _E
    pad_src = jnp.arange(npad, dtype=jnp.int32) % _N
    pad_dst = _N + jnp.arange(npad, dtype=jnp.int32) % (_NP - _N)

    def epack(v, pad):
        return jnp.concatenate(
            [v.astype(jnp.int32), pad]).reshape(_NSUB * _NCH * _SUBC, 128)

    s1p = epack(adj1[0], pad_src)
    d1p = epack(adj1[1], pad_dst)
    s2p = epack(adj2[0], pad_src)
    d2p = epack(adj2[1], pad_dst)

    _segsum_kernel, _gat_scores, _wsum = _sc_kernels()
    H0 = _h0(x, W1).reshape(2 * _N, 128)

    S1 = _segsum_kernel(H0, jnp.stack([s1p, s1p + _N]),
                        jnp.stack([d1p, d1p]))
    T1 = _segsum_kernel(H0, jnp.stack([s2p, s2p + _N]),
                        jnp.stack([d2p, d2p]))
    H1_1 = _gcn2(S1[:_N], S1[_NP:_NP + _N], b1, p1, W2)
    H1_2 = _gcn2(T1[:_N], T1[_NP:_NP + _N], b1, p1, W2)

    H1 = jnp.concatenate([H1_1, H1_2], axis=0)
    srcb = jnp.stack([s1p, s2p + _N])
    dstb = jnp.stack([d1p, d2p])
    S2 = _segsum_kernel(H1, srcb, dstb)

    hs1, HH1, AS1, AD1 = _gat_prep(S2[:_N], b2, p2, gW, att_s, att_d)
    hs2, HH2, AS2, AD2 = _gat_prep(S2[_NP:_NP + _N], b2, p2, gW, att_s,
                                   att_d)

    def pad_np(a):
        return jnp.pad(a.reshape(_N), (0, _NP - _N))

    SD, EXV, _ = _gat_scores(jnp.stack([pad_np(AS1), pad_np(AS2)]),
                             jnp.stack([pad_np(AD1), pad_np(AD2)]),
                             srcb, dstb)
    U = _wsum(jnp.concatenate([HH1, HH2], axis=0), srcb, dstb, EXV)

    hg, beta = _final(hs1, hs2, U[:_N], U[_NP:_NP + _N],
                      SD[0, :_N].reshape(_N, 1), SD[1, :_N].reshape(_N, 1),
                      deg, gb, m1W, m1b, m2W, m2b, m3W, m3b)
    return (hs1, hs2, hg, beta)


# speedup vs baseline: 30.2975x; 1.0687x over previous
import functools

import jax
import jax.numpy as jnp
from jax import lax
from jax.experimental import pallas as pl
from jax.experimental.pallas import tpu as pltpu
from jax.experimental.pallas import tpu_sc as plsc

_N = 10000
_NP = 10240
_E = 640000
_EPAD = 655360
_NSUB = 16
_CHUNK = 512
_SUBC = 4
_NCH = _EPAD // (_NSUB * _CHUNK)
_EPT = _EPAD // _NSUB
_RPS = _NP // _NSUB
_BLK = 400
_NBLK = _N // _BLK


def _h0_body(x_ref, w_ref, o_ref):
    o_ref[0] = jnp.dot(x_ref[...], w_ref[...],
                       preferred_element_type=jnp.float32)


def _h0(x, W1):
    return pl.pallas_call(
        _h0_body,
        grid=(_NBLK, 2),
        in_specs=[
            pl.BlockSpec((_BLK, 128), lambda i, j: (i, 0)),
            pl.BlockSpec((128, 128), lambda i, j: (0, j)),
        ],
        out_specs=pl.BlockSpec((1, _BLK, 128), lambda i, j: (j, i, 0)),
        out_shape=jax.ShapeDtypeStruct((2, _N, 128), jnp.float32),
    )(x, W1)


def _gcn2_body(sa_ref, sb_ref, b1a_ref, b1b_ref, p_ref, w2a_ref, w2b_ref,
               o_ref):
    p = p_ref[0, 0]
    za = sa_ref[...] + b1a_ref[...]
    ta = jnp.maximum(jnp.where(za >= 0, za, p * za), 0.0)
    zb = sb_ref[...] + b1b_ref[...]
    tb = jnp.maximum(jnp.where(zb >= 0, zb, p * zb), 0.0)
    o_ref[...] = (
        jnp.dot(ta, w2a_ref[...], preferred_element_type=jnp.float32)
        + jnp.dot(tb, w2b_ref[...], preferred_element_type=jnp.float32))


def _gcn2(sa, sb, b1, p1, W2):
    b1a = b1[:128].reshape(1, 128)
    b1b = b1[128:].reshape(1, 128)
    return pl.pallas_call(
        _gcn2_body,
        grid=(_NBLK,),
        in_specs=[
            pl.BlockSpec((_BLK, 128), lambda i: (i, 0)),
            pl.BlockSpec((_BLK, 128), lambda i: (i, 0)),
            pl.BlockSpec((1, 128), lambda i: (0, 0)),
            pl.BlockSpec((1, 128), lambda i: (0, 0)),
            pl.BlockSpec(memory_space=pltpu.SMEM),
            pl.BlockSpec((128, 128), lambda i: (0, 0)),
            pl.BlockSpec((128, 128), lambda i: (0, 0)),
        ],
        out_specs=pl.BlockSpec((_BLK, 128), lambda i: (i, 0)),
        out_shape=jax.ShapeDtypeStruct((_N, 128), jnp.float32),
    )(sa, sb, b1a, b1b, p1.reshape(1, 1), W2[:128], W2[128:])


def _gat_prep_body(s2_ref, b2_ref, p_ref, gw_ref, atts_ref, attd_ref,
                   hs_ref, hh_ref, as_ref, ad_ref):
    p = p_ref[0, 0]
    z = s2_ref[...] + b2_ref[...]
    hs = jnp.where(z >= 0, z, p * z)
    hs_ref[...] = hs
    hh = jnp.dot(hs, gw_ref[...], preferred_element_type=jnp.float32)
    hh_ref[...] = hh
    as_ref[...] = jnp.sum(hh * atts_ref[...], axis=-1, keepdims=True)
    ad_ref[...] = jnp.sum(hh * attd_ref[...], axis=-1, keepdims=True)


def _gat_prep(s2, b2, p2, gW, att_s, att_d):
    return pl.pallas_call(
        _gat_prep_body,
        grid=(_NBLK,),
        in_specs=[
            pl.BlockSpec((_BLK, 128), lambda i: (i, 0)),
            pl.BlockSpec((1, 128), lambda i: (0, 0)),
            pl.BlockSpec(memory_space=pltpu.SMEM),
            pl.BlockSpec((128, 128), lambda i: (0, 0)),
            pl.BlockSpec((1, 128), lambda i: (0, 0)),
            pl.BlockSpec((1, 128), lambda i: (0, 0)),
        ],
        out_specs=[
            pl.BlockSpec((_BLK, 128), lambda i: (i, 0)),
            pl.BlockSpec((_BLK, 128), lambda i: (i, 0)),
            pl.BlockSpec((_BLK, 1), lambda i: (i, 0)),
            pl.BlockSpec((_BLK, 1), lambda i: (i, 0)),
        ],
        out_shape=[
            jax.ShapeDtypeStruct((_N, 128), jnp.float32),
            jax.ShapeDtypeStruct((_N, 128), jnp.float32),
            jax.ShapeDtypeStruct((_N, 1), jnp.float32),
            jax.ShapeDtypeStruct((_N, 1), jnp.float32),
        ],
    )(s2, b2.reshape(1, 128), p2.reshape(1, 1), gW,
      att_s.reshape(1, 128), att_d.reshape(1, 128))


def _final_body(hs_ref, hc_ref, u1_ref, u2_ref, s1_ref, s2_ref, deg_ref,
                gb_ref, m1w_ref, m1b_ref, m2w_ref, m2b_ref, w31_ref, w32_ref,
                sc_ref, hg_ref, beta_ref):
    hs = hs_ref[...]
    hc = hc_ref[...]
    hsg = jnp.maximum(u1_ref[...] / (s1_ref[...] + 1e-16) + gb_ref[...], 0.0)
    hcg = jnp.maximum(u2_ref[...] / (s2_ref[...] + 1e-16) + gb_ref[...], 0.0)
    w3d = sc_ref[0, 0]
    m3b = sc_ref[0, 1]
    degb = deg_ref[...].astype(jnp.bfloat16).astype(jnp.float32)
    w3db = w3d.astype(jnp.bfloat16).astype(jnp.float32)

    def mlp(a, b):
        z1 = jnp.dot(a, m1w_ref[...],
                     preferred_element_type=jnp.float32) + m1b_ref[...]
        z2 = jnp.dot(b, m2w_ref[...],
                     preferred_element_type=jnp.float32) + m2b_ref[...]
        logit = (jnp.dot(z1, w31_ref[...],
                         preferred_element_type=jnp.float32)
                 + jnp.dot(z2, w32_ref[...],
                           preferred_element_type=jnp.float32)
                 + degb * w3db + m3b)
        return jax.nn.sigmoid(logit)

    beta_ref[...] = mlp(hs, hc)
    beta_g = mlp(hsg, hcg)
    hg_ref[...] = hsg + beta_g * hcg


def _final(hs, hc, u1, u2, s1, s2, deg, gb, m1W, m1b, m2W, m2b, m3W, m3b):
    sc = jnp.stack([m3W[128, 0], m3b[0]]).reshape(1, 2)
    return pl.pallas_call(
        _final_body,
        grid=(_NBLK,),
        in_specs=[
            pl.BlockSpec((_BLK, 128), lambda i: (i, 0)),
            pl.BlockSpec((_BLK, 128), lambda i: (i, 0)),
            pl.BlockSpec((_BLK, 128), lambda i: (i, 0)),
            pl.BlockSpec((_BLK, 128), lambda i: (i, 0)),
            pl.BlockSpec((_BLK, 1), lambda i: (i, 0)),
            pl.BlockSpec((_BLK, 1), lambda i: (i, 0)),
            pl.BlockSpec((_BLK, 1), lambda i: (i, 0)),
            pl.BlockSpec((1, 128), lambda i: (0, 0)),
            pl.BlockSpec((128, 64), lambda i: (0, 0)),
            pl.BlockSpec((1, 64), lambda i: (0, 0)),
            pl.BlockSpec((128, 64), lambda i: (0, 0)),
            pl.BlockSpec((1, 64), lambda i: (0, 0)),
            pl.BlockSpec((64, 1), lambda i: (0, 0)),
            pl.BlockSpec((64, 1), lambda i: (0, 0)),
            pl.BlockSpec(memory_space=pltpu.SMEM),
        ],
        out_specs=[
            pl.BlockSpec((_BLK, 128), lambda i: (i, 0)),
            pl.BlockSpec((_BLK, 1), lambda i: (i, 0)),
        ],
        out_shape=[
            jax.ShapeDtypeStruct((_N, 128), jnp.float32),
            jax.ShapeDtypeStruct((_N, 1), jnp.float32),
        ],
    )(hs, hc, u1, u2, s1, s2, deg, gb.reshape(1, 128), m1W,
      m1b.reshape(1, 64), m2W, m2b.reshape(1, 64),
      m3W[:64], m3W[64:128], sc)


def _zero_rows(rows):
    zv = jnp.zeros((16,), jnp.float32)

    def body(i, carry):
        rows[i // 8, pl.ds((i % 8) * 16, 16)] = zv
        return carry

    lax.fori_loop(0, 1024, body, 0)


_NB = _NCH * _SUBC


def _gather_scatter_pipeline(c, s, h, srcr, dstr, acc, sidx3, didx3, rows2,
                             gsem, ssem, pxs, pxd, exbf=None, exv=None,
                             pxe=None):
    base = s * _NB
    zk = jnp.zeros((16,), jnp.int32)
    pltpu.sync_copy(srcr.at[c, base], sidx3.at[0])
    pltpu.sync_copy(dstr.at[c, base], didx3.at[0])
    pltpu.async_copy(h.at[sidx3.at[0]], rows2.at[0], gsem.at[0])
    pltpu.async_copy(srcr.at[c, base + 1], sidx3.at[1], pxs.at[1])
    pltpu.async_copy(dstr.at[c, base + 1], didx3.at[1], pxd.at[1])
    if exbf is not None:
        pltpu.sync_copy(exv.at[c, base], exbf.at[pl.ds(0, 128)])
        pltpu.async_copy(exv.at[c, base + 1], exbf.at[pl.ds(128, 128)],
                         pxe.at[1])

    def body(j, carry):
        par = lax.rem(j, 2)
        nxt = 1 - par
        i3 = lax.rem(j, 3)
        n3 = lax.rem(j + 1, 3)
        p3 = lax.rem(j + 2, 3)
        pltpu.make_async_copy(h.at[sidx3.at[i3]], rows2.at[par],
                              gsem.at[par]).wait()

        @pl.when(j + 1 < _NB)
        def _issue_next_gather():
            pltpu.make_async_copy(srcr.at[c, base], sidx3.at[n3],
                                  pxs.at[n3]).wait()
            pltpu.make_async_copy(dstr.at[c, base], didx3.at[n3],
                                  pxd.at[n3]).wait()
            if exbf is not None:
                pltpu.make_async_copy(exv.at[c, base],
                                      exbf.at[pl.ds(0, 128)],
                                      pxe.at[n3]).wait()

            @pl.when(j >= 1)
            def _drain_prev_scatter():
                pltpu.make_async_copy(rows2.at[nxt], acc.at[didx3.at[i3]],
                                      ssem.at[nxt]).wait()

            pltpu.async_copy(h.at[sidx3.at[n3]], rows2.at[nxt],
                             gsem.at[nxt])

        if exbf is not None:
            off = i3 * 128

            def scale(k, carry2):
                k2 = k * 2
                w0 = plsc.load_gather(exbf, [zk + (off + k2)])
                w1 = plsc.load_gather(exbf, [zk + (off + k2 + 1)])
                for f in range(8):
                    sl = pl.ds(f * 16, 16)
                    rows2[par, k2, sl] = rows2[par, k2, sl] * w0
                    rows2[par, k2 + 1, sl] = rows2[par, k2 + 1, sl] * w1
                return carry2

            lax.fori_loop(0, 64, scale, 0)

        pltpu.make_async_copy(rows2.at[par], acc.at[didx3.at[i3]],
                              ssem.at[par]).start(add=True)

        @pl.when(j + 2 < _NB)
        def _prefetch_idx():
            pltpu.async_copy(srcr.at[c, base + j + 2], sidx3.at[p3],
                             pxs.at[p3])
            pltpu.async_copy(dstr.at[c, base + j + 2], didx3.at[p3],
                             pxd.at[p3])
            if exbf is not None:
                pltpu.async_copy(exv.at[c, base + j + 2],
                                 exbf.at[pl.ds(p3 * 128, 128)], pxe.at[p3])

        return carry

    lax.fori_loop(0, _NB, body, 0)
    for par in range(2):
        pltpu.make_async_copy(rows2.at[par], acc.at[didx3.at[0]],
                              ssem.at[par]).wait()


def _acc_zero_and_loop(c, s, h, srcr, dstr, out, sidx3, didx3, rows2, acc,
                       gsem, ssem, pxs, pxd, exbf=None, exv=None, pxe=None):
    rows0 = rows2.at[0]
    _zero_rows(rows0)
    for r in range(_RPS // 128):
        pltpu.sync_copy(rows0, acc.at[pl.ds(s * _RPS + r * 128, 128)])
    plsc.subcore_barrier()
    _gather_scatter_pipeline(c, s, h, srcr, dstr, acc, sidx3, didx3, rows2,
                             gsem, ssem, pxs, pxd, exbf, exv, pxe)
    plsc.subcore_barrier()
    for r in range(_RPS // 128):
        pltpu.sync_copy(acc.at[pl.ds(s * _RPS + r * 128, 128)], rows0)
        pltpu.sync_copy(rows0,
                        out.at[pl.ds(c * _NP + s * _RPS + r * 128, 128)])


def _segsum_body(h, srcr, dstr, out, sidx3, didx3, rows2, acc, gsem, ssem,
                 pxs, pxd):
    c = lax.axis_index("c")
    s = lax.axis_index("s")
    _acc_zero_and_loop(c, s, h, srcr, dstr, out, sidx3, didx3, rows2, acc,
                       gsem, ssem, pxs, pxd)


@functools.cache
def _sc_kernels():
    mesh = plsc.VectorSubcoreMesh(core_axis_name="c", subcore_axis_name="s")
    params = pltpu.CompilerParams(needs_layout_passes=False)
    pipe_scratch = [
        pltpu.VMEM((3, 128), jnp.int32),
        pltpu.VMEM((3, 128), jnp.int32),
        pltpu.VMEM((2, 128, 128), jnp.float32),
        pltpu.VMEM_SHARED((_NP, 128), jnp.float32),
        pltpu.SemaphoreType.DMA((2,)),
        pltpu.SemaphoreType.DMA((2,)),
        pltpu.SemaphoreType.DMA((3,)),
        pltpu.SemaphoreType.DMA((3,)),
    ]
    segsum = pl.kernel(
        _segsum_body,
        out_type=jax.ShapeDtypeStruct((2 * _NP, 128), jnp.float32),
        mesh=mesh,
        compiler_params=params,
        scratch_types=pipe_scratch,
    )
    scores = pl.kernel(
        _gat_scores_body,
        out_type=(
            jax.ShapeDtypeStruct((2, _NP), jnp.float32),
            jax.ShapeDtypeStruct((2, _NSUB * _NCH * _SUBC, 128),
                                 jnp.float32),
            jax.ShapeDtypeStruct((2, 16, _NP), jnp.float32),
        ),
        mesh=mesh,
        compiler_params=params,
        scratch_types=[
            pltpu.VMEM((2, _SUBC, 128), jnp.int32),
            pltpu.VMEM((2, _SUBC, 128), jnp.int32),
            pltpu.VMEM((_NP,), jnp.float32),
            pltpu.VMEM((_NP,), jnp.float32),
            pltpu.VMEM((_NP,), jnp.float32),
            pltpu.VMEM((256,), jnp.float32),
            pltpu.VMEM((_RPS,), jnp.float32),
            pltpu.VMEM((_RPS,), jnp.float32),
            pltpu.VMEM_SHARED((_NP,), jnp.float32),
            pltpu.SemaphoreType.DMA((2,)),
            pltpu.SemaphoreType.DMA((2,)),
            pltpu.SemaphoreType.DMA((2,)),
        ],
    )
    wsum = pl.kernel(
        _wsum_body,
        out_type=jax.ShapeDtypeStruct((2 * _NP, 128), jnp.float32),
        mesh=mesh,
        compiler_params=params,
        scratch_types=pipe_scratch[:3] + [
            pltpu.VMEM((384,), jnp.float32),
        ] + pipe_scratch[3:] + [
            pltpu.SemaphoreType.DMA((3,)),
        ],
    )
    return segsum, scores, wsum


def _gat_scores_body(asc, adc, srcr, dstr, sout, exv, mst,
                     sidx, didx, asb, adb, mb, exb, zs, macc, sden,
                     pxs, pxd, esem):
    c = lax.axis_index("c")
    s = lax.axis_index("s")
    coff = c * _N

    def chunk_loop(inner):
        base0 = s * _NCH * _SUBC
        pltpu.sync_copy(srcr.at[c, pl.ds(base0, _SUBC)], sidx.at[0])
        pltpu.sync_copy(dstr.at[c, pl.ds(base0, _SUBC)], didx.at[0])
        pltpu.async_copy(srcr.at[c, pl.ds(base0 + _SUBC, _SUBC)],
                         sidx.at[1], pxs.at[1])
        pltpu.async_copy(dstr.at[c, pl.ds(base0 + _SUBC, _SUBC)],
                         didx.at[1], pxd.at[1])

        def chunk(i, carry):
            par = lax.rem(i, 2)
            nxt = 1 - par

            @pl.when(i >= 1)
            def _wait_cur():
                pltpu.make_async_copy(srcr.at[c, pl.ds(base0, _SUBC)],
                                      sidx.at[par], pxs.at[par]).wait()
                pltpu.make_async_copy(dstr.at[c, pl.ds(base0, _SUBC)],
                                      didx.at[par], pxd.at[par]).wait()

            inner(i, par)

            @pl.when(i + 2 < _NCH)
            def _prefetch_next():
                row = base0 + (i + 2) * _SUBC
                pltpu.async_copy(srcr.at[c, pl.ds(row, _SUBC)],
                                 sidx.at[par], pxs.at[par])
                pltpu.async_copy(dstr.at[c, pl.ds(row, _SUBC)],
                                 didx.at[par], pxd.at[par])

            return carry

        lax.fori_loop(0, _NCH, chunk, 0)
    zv = jnp.zeros((16,), jnp.float32)

    def zbody(i, carry):
        zs[pl.ds(i * 16, 16)] = zv
        return carry

    lax.fori_loop(0, _RPS // 16, zbody, 0)
    pltpu.sync_copy(zs, sden.at[pl.ds(s * _RPS, _RPS)])
    ninf = jnp.full((16,), -3e38, jnp.float32)

    def mbody(i, carry):
        mb[pl.ds(i * 16, 16)] = ninf
        return carry

    lax.fori_loop(0, _NP // 16, mbody, 0)
    pltpu.sync_copy(asc.at[c], asb)
    pltpu.sync_copy(adc.at[c], adb)

    def edge_e(par, j, k):
        sv = sidx[par, j, pl.ds(k * 16, 16)] - coff
        dv = didx[par, j, pl.ds(k * 16, 16)]
        e = plsc.load_gather(asb, [sv]) + plsc.load_gather(adb, [dv])
        return jnp.where(e >= 0, e, 0.2 * e), dv

    def p1_inner(i, par):
        for j in range(_SUBC):
            def grp(k, carry2):
                e, dv = edge_e(par, j, k)

                def cond(pend):
                    return jnp.any(pend)

                def body(pend):
                    cur = plsc.load_gather(mb, [dv])
                    nm = jnp.maximum(cur, e)
                    plsc.store_scatter(mb, [dv], nm, mask=pend)
                    chk = plsc.load_gather(mb, [dv])
                    return chk < e

                lax.while_loop(cond, body, jnp.ones((16,), jnp.bool_))
                return carry2

            lax.fori_loop(0, 8, grp, 0)

    chunk_loop(p1_inner)

    pltpu.sync_copy(mb, mst.at[c, s])
    plsc.subcore_barrier()
    sl = pl.ds(s * _RPS, _RPS)
    pltpu.sync_copy(mst.at[c, 0, sl], macc)

    def redmax(r, carry):
        pltpu.sync_copy(mst.at[c, r, sl], zs)

        def vmax(i, carry2):
            v = pl.ds(i * 16, 16)
            macc[v] = jnp.maximum(macc[v], zs[v])
            return carry2

        lax.fori_loop(0, _RPS // 16, vmax, 0)
        return carry

    lax.fori_loop(1, 16, redmax, 0)

    def fin(i, carry):
        v = pl.ds(i * 16, 16)
        m = macc[v]
        macc[v] = jnp.where(m < -2.9e38, 0.0, m)
        return carry

    lax.fori_loop(0, _RPS // 16, fin, 0)
    pltpu.sync_copy(macc, mst.at[c, 0, sl])
    plsc.subcore_barrier()
    pltpu.sync_copy(mst.at[c, 0], mb)

    def p3_inner(i, par):
        row0 = (s * _NCH + i) * _SUBC
        for j in range(_SUBC):
            jp = j % 2
            sl = pl.ds(jp * 128, 128)

            def drain():
                pltpu.make_async_copy(exb.at[sl], exv.at[c, row0],
                                      esem.at[jp]).wait()

            if j < 2:
                pl.when(i >= 1)(drain)
            else:
                drain()

            def grp(k, carry2):
                e, dv = edge_e(par, j, k)
                mg = plsc.load_gather(mb, [dv])
                exb[pl.ds(jp * 128 + k * 16, 16)] = jnp.exp(e - mg)
                return carry2

            lax.fori_loop(0, 8, grp, 0)
            pltpu.make_async_copy(exb.at[sl], exv.at[c, row0 + j],
                                  esem.at[jp]).start()
            pltpu.sync_copy(exb.at[sl], sden.at[didx.at[par, j]], add=True)

    chunk_loop(p3_inner)
    for jp in range(2):
        pltpu.make_async_copy(exb.at[pl.ds(jp * 128, 128)],
                              exv.at[c, jp], esem.at[jp]).wait()
    plsc.subcore_barrier()
    pltpu.sync_copy(sden.at[pl.ds(s * _RPS, _RPS)], zs)
    pltpu.sync_copy(zs, sout.at[c, pl.ds(s * _RPS, _RPS)])


def _wsum_body(h, srcr, dstr, exv, out, sidx3, didx3, rows2, exbf, acc,
               gsem, ssem, pxs, pxd, pxe):
    c = lax.axis_index("c")
    s = lax.axis_index("s")
    _acc_zero_and_loop(c, s, h, srcr, dstr, out, sidx3, didx3, rows2, acc,
                       gsem, ssem, pxs, pxd, exbf, exv, pxe)


def kernel(x, adj1, adj2, deg, W1, b1, p1, W2, b2, p2, gW, att_s, att_d, gb,
           m1W, m1b, m2W, m2b, m3W, m3b):
    npad = _EPAD - _E
    pad_src = jnp.arange(npad, dtype=jnp.int32) % _N
    pad_dst = _N + jnp.arange(npad, dtype=jnp.int32) % (_NP - _N)

    def epack(v, pad):
        return jnp.concatenate(
            [v.astype(jnp.int32), pad]).reshape(_NSUB * _NCH * _SUBC, 128)

    s1p = epack(adj1[0], pad_src)
    d1p = epack(adj1[1], pad_dst)
    s2p = epack(adj2[0], pad_src)
    d2p = epack(adj2[1], pad_dst)

    _segsum_kernel, _gat_scores, _wsum = _sc_kernels()
    H0 = _h0(x, W1).reshape(2 * _N, 128)

    S1 = _segsum_kernel(H0, jnp.stack([s1p, s1p + _N]),
                        jnp.stack([d1p, d1p]))
    T1 = _segsum_kernel(H0, jnp.stack([s2p, s2p + _N]),
                        jnp.stack([d2p, d2p]))
    H1_1 = _gcn2(S1[:_N], S1[_NP:_NP + _N], b1, p1, W2)
    H1_2 = _gcn2(T1[:_N], T1[_NP:_NP + _N], b1, p1, W2)

    H1 = jnp.concatenate([H1_1, H1_2], axis=0)
    srcb = jnp.stack([s1p, s2p + _N])
    dstb = jnp.stack([d1p, d2p])
    S2 = _segsum_kernel(H1, srcb, dstb)

    hs1, HH1, AS1, AD1 = _gat_prep(S2[:_N], b2, p2, gW, att_s, att_d)
    hs2, HH2, AS2, AD2 = _gat_prep(S2[_NP:_NP + _N], b2, p2, gW, att_s,
                                   att_d)

    def pad_np(a):
        return jnp.pad(a.reshape(_N), (0, _NP - _N))

    SD, EXV, _ = _gat_scores(jnp.stack([pad_np(AS1), pad_np(AS2)]),
                             jnp.stack([pad_np(AD1), pad_np(AD2)]),
                             srcb, dstb)
    U = _wsum(jnp.concatenate([HH1, HH2], axis=0), srcb, dstb, EXV)

    hg, beta = _final(hs1, hs2, U[:_N], U[_NP:_NP + _N],
                      SD[0, :_N].reshape(_N, 1), SD[1, :_N].reshape(_N, 1),
                      deg, gb, m1W, m1b, m2W, m2b, m3W, m3b)
    return (hs1, hs2, hg, beta)
